# Initial kernel scaffold; baseline (speedup 1.0000x reference)
import jax
import jax.numpy as jnp
from jax.experimental import pallas as pl


def kernel(x, drug_drug_batch, edge_attr, edge_cell_lines, sg_edge_index, sg_nodes, sg_avging_idx, W1, b1, W2, b2):
    raise NotImplementedError("write your pallas kernel here")


# speedup vs baseline: 10.8438x; 10.8438x over previous
import functools

import jax
import jax.numpy as jnp
from jax import lax
from jax.experimental import pallas as pl
from jax.experimental.pallas import tpu as pltpu
from jax.experimental.pallas import tpu_sc as plsc

N_NODES = 10000
N_EDGES = 320000
D = 128
N_DRUGS = 1024
N_SG = 102400
N_DD = 4096

NC = 2
NS = 16
NW = NC * NS
CH = 128

EPT = N_EDGES // NW
EG = 80
EPTP = EG * CH
NPAD = 10240
ZROW = NPAD // NS

SG_PT = N_SG // NW
SG_G = SG_PT // CH
DD_PT = N_DD // NW

_mesh = plsc.VectorSubcoreMesh(core_axis_name="c", subcore_axis_name="s")


def _wid():
    c = lax.axis_index("c")
    s = lax.axis_index("s")
    return c, s, c * NS + s


def _hist_body(dstp, sga, deg_out, cnt_out,
               acc_deg, acc_cnt, dbuf, abuf, ones, zbuf, sem):
    c, s, w = _wid()
    z16 = jnp.zeros((16,), jnp.float32)

    def zb(i, _):
        zbuf[pl.ds(i * 16, 16)] = z16
        return 0
    lax.fori_loop(0, ZROW // 16, zb, 0)
    pltpu.sync_copy(zbuf, acc_deg.at[pl.ds(s * ZROW, ZROW)])
    pltpu.sync_copy(zbuf.at[pl.ds(0, N_DRUGS // NS)],
                    acc_cnt.at[pl.ds(s * (N_DRUGS // NS), N_DRUGS // NS)])
    for i in range(CH // 16):
        ones[pl.ds(16 * i, 16)] = jnp.ones((16,), jnp.float32)
    plsc.subcore_barrier()

    pltpu.sync_copy(dstp.at[w], dbuf)
    pltpu.sync_copy(sga.at[w], abuf)
    for base in range(0, EG, 16):
        descs = [pltpu.async_copy(ones, acc_deg.at[dbuf.at[base + j]], sem,
                                  add=True) for j in range(16)]
        for d in descs:
            d.wait()
    for base in range(0, SG_G, 13):
        n = min(13, SG_G - base)
        descs = [pltpu.async_copy(ones, acc_cnt.at[abuf.at[base + j]], sem,
                                  add=True) for j in range(n)]
        for d in descs:
            d.wait()
    plsc.subcore_barrier()
    @pl.when(s == 0)
    def _():
        pltpu.sync_copy(acc_deg, deg_out.at[c])
    @pl.when(s == 1)
    def _():
        pltpu.sync_copy(acc_cnt, cnt_out.at[c])


def _make_hist():
    return pl.kernel(
        _hist_body,
        out_type=[jax.ShapeDtypeStruct((NC, NPAD), jnp.float32),
                  jax.ShapeDtypeStruct((NC, N_DRUGS), jnp.float32)],
        mesh=_mesh,
        scratch_types=[
            pltpu.VMEM_SHARED((NPAD,), jnp.float32),
            pltpu.VMEM_SHARED((N_DRUGS,), jnp.float32),
            pltpu.VMEM((EG, CH), jnp.int32),
            pltpu.VMEM((SG_G, CH), jnp.int32),
            pltpu.VMEM((CH,), jnp.float32),
            pltpu.VMEM((ZROW,), jnp.float32),
            pltpu.SemaphoreType.DMA,
        ],
    )


def _zero_rows(rows_ref):
    z16 = jnp.zeros((16,), jnp.float32)

    def zb(r, _):
        for j in range(D // 16):
            rows_ref[r, pl.ds(j * 16, 16)] = z16
        return 0
    lax.fori_loop(0, CH, zb, 0)


def _agg_body(y, sd, out0, out1,
              acc, ib0, ib1, rows0, rows1, gsem, ssem, isem):
    c, s, w = _wid()
    r0 = s * ZROW
    _zero_rows(rows0)
    for j in range(ZROW // CH):
        pltpu.sync_copy(rows0, acc.at[pl.ds(r0 + j * CH, CH)])
    plsc.subcore_barrier()

    rows = [rows0, rows1]
    ibs = [ib0, ib1]
    base = w * EG
    pend_i = [pltpu.async_copy(sd.at[base], ib0, isem), None]
    if EG > 1:
        pend_i[1] = pltpu.async_copy(sd.at[base + 1], ib1, isem)
    pend_i[0].wait()
    pend_g = pltpu.async_copy(y.at[ib0.at[0]], rows0, gsem)
    for g in range(EG):
        ib = ibs[g % 2]
        cur = rows[g % 2]
        pend_g.wait()
        if g + 1 < EG:
            ibn = ibs[(g + 1) % 2]
            pend_i[(g + 1) % 2].wait()
            pend_g = pltpu.async_copy(y.at[ibn.at[0]], rows[(g + 1) % 2],
                                      gsem)
        pltpu.async_copy(cur, acc.at[ib.at[1]], ssem, add=True).wait()
        if g + 2 < EG:
            pend_i[g % 2] = pltpu.async_copy(sd.at[base + g + 2], ib, isem)
    plsc.subcore_barrier()

    for core, out in ((0, out0), (1, out1)):
        @pl.when(c == core)
        def _():
            @pl.when(s < NS - 1)
            def _():
                pltpu.sync_copy(acc.at[pl.ds(r0, ZROW)],
                                out.at[pl.ds(r0, ZROW)])
            @pl.when(s == NS - 1)
            def _():
                last = N_NODES - (NS - 1) * ZROW
                pltpu.sync_copy(acc.at[pl.ds((NS - 1) * ZROW, last)],
                                out.at[pl.ds((NS - 1) * ZROW, last)])


def _make_agg():
    return pl.kernel(
        _agg_body,
        out_type=[jax.ShapeDtypeStruct((N_NODES, D), jnp.float32),
                  jax.ShapeDtypeStruct((N_NODES, D), jnp.float32)],
        mesh=_mesh,
        scratch_types=[
            pltpu.VMEM_SHARED((NPAD, D), jnp.float32),
            pltpu.VMEM((2, CH), jnp.int32),
            pltpu.VMEM((2, CH), jnp.int32),
            pltpu.VMEM((CH, D), jnp.float32),
            pltpu.VMEM((CH, D), jnp.float32),
            pltpu.SemaphoreType.DMA,
            pltpu.SemaphoreType.DMA,
            pltpu.SemaphoreType.DMA,
        ],
    )


def _pool_body(h2, sgn, sga, out0, out1,
               acc, nbuf, abuf, rows0, rows1, gsem, ssem):
    c, s, w = _wid()
    rp = N_DRUGS // NS
    _zero_rows(rows0)
    pltpu.sync_copy(rows0.at[pl.ds(0, rp)], acc.at[pl.ds(s * rp, rp)])
    plsc.subcore_barrier()

    pltpu.sync_copy(sgn.at[w], nbuf)
    pltpu.sync_copy(sga.at[w], abuf)
    rows = [rows0, rows1]
    pend = pltpu.async_copy(h2.at[nbuf.at[0]], rows0, gsem)
    for g in range(SG_G):
        cur = rows[g % 2]
        pend.wait()
        if g + 1 < SG_G:
            pend = pltpu.async_copy(h2.at[nbuf.at[g + 1]], rows[(g + 1) % 2],
                                    gsem)
        pltpu.async_copy(cur, acc.at[abuf.at[g]], ssem, add=True).wait()
    plsc.subcore_barrier()
    for core, out in ((0, out0), (1, out1)):
        @pl.when(c == core)
        def _():
            pltpu.sync_copy(acc.at[pl.ds(s * rp, rp)],
                            out.at[pl.ds(s * rp, rp)])


def _make_pool():
    return pl.kernel(
        _pool_body,
        out_type=[jax.ShapeDtypeStruct((N_DRUGS, D), jnp.float32),
                  jax.ShapeDtypeStruct((N_DRUGS, D), jnp.float32)],
        mesh=_mesh,
        scratch_types=[
            pltpu.VMEM_SHARED((N_DRUGS, D), jnp.float32),
            pltpu.VMEM((SG_G, CH), jnp.int32),
            pltpu.VMEM((SG_G, CH), jnp.int32),
            pltpu.VMEM((CH, D), jnp.float32),
            pltpu.VMEM((CH, D), jnp.float32),
            pltpu.SemaphoreType.DMA,
            pltpu.SemaphoreType.DMA,
        ],
    )


_PB = 512


def _tc_pred_body(p0_ref, p1_ref, c0_ref, c1_ref, ai_ref, bi_ref, out_ref):
    cnt = jnp.maximum(c0_ref[...] + c1_ref[...], 1.0)
    emb = (p0_ref[...] + p1_ref[...]) / cnt
    ids = lax.broadcasted_iota(jnp.int32, (_PB, N_DRUGS), 1)
    oh_a = (ai_ref[...] == ids).astype(jnp.float32)
    oh_b = (bi_ref[...] == ids).astype(jnp.float32)
    t1 = jnp.dot(oh_a, emb, preferred_element_type=jnp.float32)
    t2 = jnp.dot(oh_b, emb, preferred_element_type=jnp.float32)
    out_ref[...] = jnp.sum(t1 * t2, axis=1, keepdims=True)


def _tc_pred(pool0, pool1, cnt0c, cnt1c, ai, bi):
    return pl.pallas_call(
        _tc_pred_body,
        grid=(N_DD // _PB,),
        in_specs=[
            pl.BlockSpec((N_DRUGS, D), lambda i: (0, 0)),
            pl.BlockSpec((N_DRUGS, D), lambda i: (0, 0)),
            pl.BlockSpec((N_DRUGS, 1), lambda i: (0, 0)),
            pl.BlockSpec((N_DRUGS, 1), lambda i: (0, 0)),
            pl.BlockSpec((_PB, 1), lambda i: (i, 0)),
            pl.BlockSpec((_PB, 1), lambda i: (i, 0)),
        ],
        out_specs=pl.BlockSpec((_PB, 1), lambda i: (i, 0)),
        out_shape=jax.ShapeDtypeStruct((N_DD, 1), jnp.float32),
    )(pool0, pool1, cnt0c, cnt1c, ai, bi)


_RB = 1000
_GRID = N_NODES // _RB


def _tc_a_body(x_ref, w1_ref, degt_ref, y1_ref, dinv_ref):
    xw = jnp.dot(x_ref[...], w1_ref[...], preferred_element_type=jnp.float32)
    deg = degt_ref[:, 0:1] + degt_ref[:, 1:2] + 1.0
    dinv = lax.rsqrt(jnp.maximum(deg, 1.0))
    dinv_ref[...] = dinv
    y1_ref[...] = xw * dinv


def _tc_a(x, w1, degt):
    return pl.pallas_call(
        _tc_a_body,
        grid=(_GRID,),
        in_specs=[
            pl.BlockSpec((_RB, D), lambda i: (i, 0)),
            pl.BlockSpec((D, D), lambda i: (0, 0)),
            pl.BlockSpec((_RB, 2), lambda i: (i, 0)),
        ],
        out_specs=[
            pl.BlockSpec((_RB, D), lambda i: (i, 0)),
            pl.BlockSpec((_RB, 1), lambda i: (i, 0)),
        ],
        out_shape=[jax.ShapeDtypeStruct((N_NODES, D), jnp.float32),
                   jax.ShapeDtypeStruct((N_NODES, 1), jnp.float32)],
    )(x, w1, degt)


def _tc_b_body(p0_ref, p1_ref, y1_ref, dinv_ref, b1_ref, w2_ref,
               h1_ref, y2_ref):
    dinv = dinv_ref[...]
    pre = dinv * (p0_ref[...] + p1_ref[...] + y1_ref[...]) + b1_ref[...]
    h1 = jnp.maximum(pre, 0.0)
    h1_ref[...] = h1
    xw2 = jnp.dot(h1, w2_ref[...], preferred_element_type=jnp.float32)
    y2_ref[...] = xw2 * dinv


def _tc_b(p0, p1, y1, dinv, b1, w2):
    return pl.pallas_call(
        _tc_b_body,
        grid=(_GRID,),
        in_specs=[
            pl.BlockSpec((_RB, D), lambda i: (i, 0)),
            pl.BlockSpec((_RB, D), lambda i: (i, 0)),
            pl.BlockSpec((_RB, D), lambda i: (i, 0)),
            pl.BlockSpec((_RB, 1), lambda i: (i, 0)),
            pl.BlockSpec((1, D), lambda i: (0, 0)),
            pl.BlockSpec((D, D), lambda i: (0, 0)),
        ],
        out_specs=[
            pl.BlockSpec((_RB, D), lambda i: (i, 0)),
            pl.BlockSpec((_RB, D), lambda i: (i, 0)),
        ],
        out_shape=[jax.ShapeDtypeStruct((N_NODES, D), jnp.float32),
                   jax.ShapeDtypeStruct((N_NODES, D), jnp.float32)],
    )(p0, p1, y1, dinv, b1, w2)


def _tc_c_body(p0_ref, p1_ref, y2_ref, dinv_ref, b2_ref, h1_ref, h2_ref):
    pre = (dinv_ref[...] * (p0_ref[...] + p1_ref[...] + y2_ref[...])
           + b2_ref[...])
    h2_ref[...] = jnp.maximum(pre, 0.0) + h1_ref[...]


def _tc_c(p0, p1, y2, dinv, b2, h1):
    return pl.pallas_call(
        _tc_c_body,
        grid=(_GRID,),
        in_specs=[
            pl.BlockSpec((_RB, D), lambda i: (i, 0)),
            pl.BlockSpec((_RB, D), lambda i: (i, 0)),
            pl.BlockSpec((_RB, D), lambda i: (i, 0)),
            pl.BlockSpec((_RB, 1), lambda i: (i, 0)),
            pl.BlockSpec((1, D), lambda i: (0, 0)),
            pl.BlockSpec((_RB, D), lambda i: (i, 0)),
        ],
        out_specs=pl.BlockSpec((_RB, D), lambda i: (i, 0)),
        out_shape=jax.ShapeDtypeStruct((N_NODES, D), jnp.float32),
    )(p0, p1, y2, dinv, b2, h1)


def kernel(x, drug_drug_batch, edge_attr, edge_cell_lines, sg_edge_index,
           sg_nodes, sg_avging_idx, W1, b1, W2, b2):
    i32 = jnp.int32
    src = sg_edge_index[0].astype(i32).reshape(NW, EPT)
    dst = sg_edge_index[1].astype(i32).reshape(NW, EPT)
    npad = EPTP - EPT
    pad_src = jnp.zeros((NW, npad), i32)
    pad_dst = jnp.broadcast_to(
        N_NODES + (jnp.arange(npad, dtype=i32) % (NPAD - N_NODES)),
        (NW, npad))
    srcp = jnp.concatenate([src, pad_src], axis=1).reshape(NW, EG, CH)
    dstp = jnp.concatenate([dst, pad_dst], axis=1).reshape(NW, EG, CH)
    sd = jnp.stack([srcp, dstp], axis=2).reshape(NW * EG, 2, CH)
    sgn = sg_nodes.astype(i32).reshape(NW, SG_G, CH)
    sga = sg_avging_idx.astype(i32).reshape(NW, SG_G, CH)
    ai = drug_drug_batch[0].astype(i32).reshape(N_DD, 1)
    bi = drug_drug_batch[1].astype(i32).reshape(N_DD, 1)
    deg_p, cnt_p = _make_hist()(dstp, sga)
    degt = jnp.stack([deg_p[0, :N_NODES], deg_p[1, :N_NODES]], axis=1)

    y1, dinv = _tc_a(x, W1, degt)
    p1_0, p1_1 = _make_agg()(y1, sd)
    h1, y2 = _tc_b(p1_0, p1_1, y1, dinv, b1.reshape(1, D), W2)
    p2_0, p2_1 = _make_agg()(y2, sd)
    h2 = _tc_c(p2_0, p2_1, y2, dinv, b2.reshape(1, D), h1)

    pool0, pool1 = _make_pool()(h2, sgn, sga)
    pred = _tc_pred(pool0, pool1, cnt_p[0].reshape(N_DRUGS, 1),
                    cnt_p[1].reshape(N_DRUGS, 1), ai, bi)
    return pred.reshape(N_DD)
